# baseline (speedup 1.0000x reference)
import functools

import jax
import jax.numpy as jnp
from jax import lax
from jax.experimental import pallas as pl
from jax.experimental.pallas import tpu as pltpu
from jax.experimental.pallas import tpu_sc as plsc

DMODEL = 64
SCALE = 8.0
C = 128
NBUF = 4


def _make_sc_embed(nw, nc, ch):
    mesh = plsc.VectorSubcoreMesh(core_axis_name="c", subcore_axis_name="s")

    @functools.partial(
        pl.kernel,
        mesh=mesh,
        compiler_params=pltpu.CompilerParams(use_tc_tiling_on_sc=False),
        out_type=jax.ShapeDtypeStruct((nw, ch, C, DMODEL), jnp.float32),
        scratch_types=(
            [pltpu.VMEM((ch, C), jnp.int32)]
            + [pltpu.VMEM((C, DMODEL), jnp.float32) for _ in range(NBUF)]
            + [pltpu.SemaphoreType.DMA for _ in range(2 * NBUF)]
        ),
    )
    def emb(x_hbm, table_hbm, out_hbm, idx_v, *rest):
        bufs = rest[:NBUF]
        gsems = rest[NBUF:2 * NBUF]
        osems = rest[2 * NBUF:]
        wid = lax.axis_index("s") * nc + lax.axis_index("c")

        pltpu.sync_copy(x_hbm.at[wid], idx_v)

        for b in range(NBUF):
            pltpu.async_copy(table_hbm.at[idx_v.at[b]], bufs[b], gsems[b])

        def step(j, carry):
            kb = j * NBUF
            for b in range(NBUF):
                k = kb + b
                buf = bufs[b]
                pltpu.make_async_copy(
                    table_hbm.at[idx_v.at[k]], buf, gsems[b]).wait()

                def row_body(i, c2, buf=buf):
                    for cc in range(DMODEL // 16):
                        sl = pl.ds(cc * 16, 16)
                        buf[i, sl] = buf[i, sl] * SCALE
                    return c2

                lax.fori_loop(0, C, row_body, 0)
                pltpu.async_copy(buf, out_hbm.at[wid, k], osems[b])

                @pl.when(k + NBUF < ch)
                def _(b=b, k=k, buf=buf):
                    pltpu.make_async_copy(
                        buf, out_hbm.at[wid, k], osems[b]).wait()
                    pltpu.async_copy(
                        table_hbm.at[idx_v.at[k + NBUF]], bufs[b], gsems[b])
            return carry

        lax.fori_loop(0, ch // NBUF, step, 0)

        for b in range(NBUF):
            pltpu.make_async_copy(bufs[b], out_hbm.at[wid, 0], osems[b]).wait()

    return emb


def kernel(x, table):
    S, T = x.shape
    B = S * T
    info = plsc.get_sparse_core_info()
    nc, ns = info.num_cores, info.num_subcores
    nw = nc * ns
    ch = B // (nw * C)
    x3 = x.astype(jnp.int32).reshape(nw, ch, C)
    out = _make_sc_embed(nw, nc, ch)(x3, table)
    return out.reshape(S, T, DMODEL)
